# Initial kernel scaffold; baseline (speedup 1.0000x reference)
import jax
import jax.numpy as jnp
from jax.experimental import pallas as pl


def kernel(node_feature, geo_encoding, edge_index, is_source, W0, b0, W1, b1, W2, b2, W3, b3, att):
    raise NotImplementedError("write your pallas kernel here")


# speedup vs baseline: 3.2523x; 3.2523x over previous
import functools

import jax
import jax.numpy as jnp
from jax import lax
from jax.experimental import pallas as pl
from jax.experimental.pallas import tpu as pltpu
from jax.experimental.pallas import tpu_sc as plsc

N = 10000
E = 320000
D = 128
G = 13
DH = D // 2

NC = 2
NS = 16
LANES = 16
NW = NC * NS
EPT = E // NW
EPT2 = E // NS
CH = 400
SUB = 80
NSUB = CH // SUB
ROWA = 624
TAIL = N - NS * ROWA

_MESH = plsc.VectorSubcoreMesh(
    core_axis_name="c", subcore_axis_name="s", num_cores=NC, num_subcores=NS
)
_SC_PARAMS = pltpu.CompilerParams(use_tc_tiling_on_sc=False)


@functools.partial(
    pl.kernel,
    out_type=(
        jax.ShapeDtypeStruct((E, D), jnp.float32),
        jax.ShapeDtypeStruct((NC, E, DH), jnp.float32),
    ),
    mesh=_MESH,
    scratch_types=[
        pltpu.VMEM((CH,), jnp.int32),
        pltpu.VMEM((CH,), jnp.int32),
        pltpu.VMEM((CH, D), jnp.float32),
        pltpu.VMEM((CH, DH), jnp.float32),
        pltpu.VMEM((CH, DH), jnp.float32),
        pltpu.SemaphoreType.DMA,
        pltpu.SemaphoreType.DMA,
        pltpu.SemaphoreType.DMA,
    ],
    compiler_params=_SC_PARAMS,
)
def _gather_k(nf, nf_lo, nf_hi, dst, src, fi_out, fj2_out,
              idx_i, idx_j, buf_i, buf_lo, buf_hi, sem_i, sem_lo, sem_hi):
    wid = lax.axis_index("s") * NC + lax.axis_index("c")
    base0 = wid * EPT

    def body(c, carry):
        base = base0 + c * CH
        pltpu.sync_copy(dst.at[pl.ds(base, CH)], idx_i)
        pltpu.sync_copy(src.at[pl.ds(base, CH)], idx_j)
        cp_i = pltpu.async_copy(nf.at[idx_i], buf_i, sem_i)
        cp_lo = pltpu.async_copy(nf_lo.at[idx_j], buf_lo, sem_lo)
        cp_hi = pltpu.async_copy(nf_hi.at[idx_j], buf_hi, sem_hi)
        cp_i.wait()
        cp_lo.wait()
        cp_hi.wait()
        pltpu.sync_copy(buf_i, fi_out.at[pl.ds(base, CH)])
        pltpu.sync_copy(buf_lo, fj2_out.at[0, pl.ds(base, CH)])
        pltpu.sync_copy(buf_hi, fj2_out.at[1, pl.ds(base, CH)])
        return carry

    lax.fori_loop(0, EPT // CH, body, 0)


BLK = 1280
NBLK = E // BLK


def _mlp_body(fi, fj2, geo, w0a, w0bl, w0bh, w0c, b0, w1, b1, w2, b2, w3, b3,
              att, alpha_ref, m_ref, mscr):
    step = pl.program_id(0)

    @pl.when(step == 0)
    def _():
        mscr[0] = -jnp.inf

    h = (
        jnp.dot(fi[...], w0a[...], preferred_element_type=jnp.float32)
        + jnp.dot(fj2[0], w0bl[...], preferred_element_type=jnp.float32)
        + jnp.dot(fj2[1], w0bh[...], preferred_element_type=jnp.float32)
        + jnp.dot(geo[...], w0c[...], preferred_element_type=jnp.float32)
        + b0[...]
    )
    h = jnp.maximum(h, 0.0)
    h = jnp.maximum(jnp.dot(h, w1[...], preferred_element_type=jnp.float32) + b1[...], 0.0)
    h = jnp.maximum(jnp.dot(h, w2[...], preferred_element_type=jnp.float32) + b2[...], 0.0)
    h = jnp.maximum(jnp.dot(h, w3[...], preferred_element_type=jnp.float32) + b3[...], 0.0)
    y = h * att[...]
    y = jnp.where(y >= 0.0, y, 0.01 * y)
    a = jnp.sum(y, axis=1)
    alpha_ref[0, 0, :] = a
    mscr[0] = jnp.maximum(mscr[0], jnp.max(a))

    @pl.when(step == NBLK - 1)
    def _():
        m_ref[...] = jnp.full((1, D), mscr[0], jnp.float32)


def _run_mlp(fi_g, fj2_g, geo, w0a, w0bl, w0bh, w0c, b0, w1, b1, w2, b2, w3,
             b3, att):
    full = lambda shape: pl.BlockSpec(shape, lambda i: (0,) * len(shape))
    alpha3, m2d = pl.pallas_call(
        _mlp_body,
        grid=(NBLK,),
        in_specs=[
            pl.BlockSpec((BLK, D), lambda i: (i, 0)),
            pl.BlockSpec((NC, BLK, DH), lambda i: (0, i, 0)),
            pl.BlockSpec((BLK, G), lambda i: (i, 0)),
            full((D, D)), full((DH, D)), full((DH, D)), full((G, D)),
            full((1, D)),
            full((D, D)), full((1, D)),
            full((D, D)), full((1, D)),
            full((D, D)), full((1, D)),
            full((1, D)),
        ],
        out_specs=[
            pl.BlockSpec((1, 1, BLK), lambda i: (i, 0, 0)),
            pl.BlockSpec((1, D), lambda i: (0, 0)),
        ],
        out_shape=[
            jax.ShapeDtypeStruct((NBLK, 1, BLK), jnp.float32),
            jax.ShapeDtypeStruct((1, D), jnp.float32),
        ],
        scratch_shapes=[pltpu.SMEM((1,), jnp.float32)],
    )(fi_g, fj2_g, geo, w0a, w0bl, w0bh, w0c, b0, w1, b1, w2, b2, w3, b3, att)
    return alpha3, m2d


@functools.partial(
    pl.kernel,
    out_type=(
        jax.ShapeDtypeStruct((NC, N, DH), jnp.float32),
        jax.ShapeDtypeStruct((NC, N, LANES), jnp.float32),
    ),
    mesh=_MESH,
    scratch_types=[
        pltpu.VMEM((CH, DH), jnp.float32),
        pltpu.VMEM((CH, LANES), jnp.float32),
        pltpu.VMEM((CH,), jnp.float32),
        pltpu.VMEM((NSUB, SUB), jnp.int32),
        pltpu.VMEM((LANES,), jnp.float32),
        pltpu.VMEM_SHARED((N, DH), jnp.float32),
        pltpu.VMEM_SHARED((N, LANES), jnp.float32),
    ],
    compiler_params=_SC_PARAMS,
)
def _scatter_k(fj2, alpha, m_arr, dst, acc_out, den_out,
               buf, dstage, aw, idxw, mv, acc_sh, den_sh):
    cid = lax.axis_index("c")
    sid = lax.axis_index("s")
    base0 = sid * EPT2

    pltpu.sync_copy(m_arr.at[pl.ds(0, LANES)], mv)
    zv = jnp.zeros((LANES,), jnp.float32)

    def zrow(r, carry):
        for c in range(DH // LANES):
            buf[r, pl.ds(c * LANES, LANES)] = zv
        dstage[r, pl.ds(0, LANES)] = zv
        return carry

    lax.fori_loop(0, CH, zrow, 0)
    rows0 = sid * ROWA
    pltpu.sync_copy(buf, acc_sh.at[pl.ds(rows0, CH)])
    pltpu.sync_copy(buf.at[pl.ds(0, ROWA - CH)], acc_sh.at[pl.ds(rows0 + CH, ROWA - CH)])
    pltpu.sync_copy(dstage, den_sh.at[pl.ds(rows0, CH)])
    pltpu.sync_copy(dstage.at[pl.ds(0, ROWA - CH)], den_sh.at[pl.ds(rows0 + CH, ROWA - CH)])

    @pl.when(sid == NS - 1)
    def _():
        pltpu.sync_copy(buf.at[pl.ds(0, TAIL)], acc_sh.at[pl.ds(NS * ROWA, TAIL)])
        pltpu.sync_copy(dstage.at[pl.ds(0, TAIL)], den_sh.at[pl.ds(NS * ROWA, TAIL)])

    plsc.subcore_barrier()

    def chunk(c, carry):
        base = base0 + c * CH
        pltpu.sync_copy(fj2.at[cid, pl.ds(base, CH)], buf)
        pltpu.sync_copy(alpha.at[pl.ds(base, CH)], aw)
        for t in range(NSUB):
            pltpu.sync_copy(dst.at[pl.ds(base + t * SUB, SUB)], idxw.at[t])

        def grp(g, carry2):
            av = aw[pl.ds(g * LANES, LANES)]
            ex = jnp.exp(av - mv[...])
            for l in range(LANES):
                b = jnp.broadcast_to(ex[l], (LANES,))
                r = g * LANES + l
                dstage[r, pl.ds(0, LANES)] = b
                for c2 in range(DH // LANES):
                    sl = pl.ds(c2 * LANES, LANES)
                    buf[r, sl] = buf[r, sl] * b
            return carry2

        lax.fori_loop(0, CH // LANES, grp, 0)

        for t in range(NSUB):
            pltpu.sync_copy(buf.at[pl.ds(t * SUB, SUB)],
                            acc_sh.at[idxw.at[t]], add=True)

        @pl.when(cid == 0)
        def _():
            for t in range(NSUB):
                pltpu.sync_copy(dstage.at[pl.ds(t * SUB, SUB)],
                                den_sh.at[idxw.at[t]], add=True)

        return carry

    lax.fori_loop(0, EPT2 // CH, chunk, 0)
    plsc.subcore_barrier()

    pltpu.sync_copy(acc_sh.at[pl.ds(rows0, ROWA)],
                    acc_out.at[cid, pl.ds(rows0, ROWA)])
    pltpu.sync_copy(den_sh.at[pl.ds(rows0, ROWA)],
                    den_out.at[cid, pl.ds(rows0, ROWA)])

    @pl.when(sid == NS - 1)
    def _():
        pltpu.sync_copy(acc_sh.at[pl.ds(NS * ROWA, TAIL)],
                        acc_out.at[cid, pl.ds(NS * ROWA, TAIL)])
        pltpu.sync_copy(den_sh.at[pl.ds(NS * ROWA, TAIL)],
                        den_out.at[cid, pl.ds(NS * ROWA, TAIL)])


BN = 2000


def _combine_body(nf, acc, den, out):
    a = jnp.concatenate([acc[0], acc[1]], axis=-1)
    d = den[0, :, 0:1] + den[1, :, 0:1]
    d = jnp.where(d > 0.0, d, 1.0)
    out[...] = nf[...] + a / d


def _run_combine(node_feature, acc, den):
    return pl.pallas_call(
        _combine_body,
        grid=(N // BN,),
        in_specs=[
            pl.BlockSpec((BN, D), lambda i: (i, 0)),
            pl.BlockSpec((NC, BN, DH), lambda i: (0, i, 0)),
            pl.BlockSpec((NC, BN, LANES), lambda i: (0, i, 0)),
        ],
        out_specs=pl.BlockSpec((BN, D), lambda i: (i, 0)),
        out_shape=jax.ShapeDtypeStruct((N, D), jnp.float32),
    )(node_feature, acc, den)


def kernel(node_feature, geo_encoding, edge_index, is_source,
           W0, b0, W1, b1, W2, b2, W3, b3, att):
    del is_source
    w0t = W0.T
    w0a = w0t[:D]
    w0bl = w0t[D:D + DH]
    w0bh = w0t[D + DH:2 * D]
    w0c = w0t[2 * D:]
    src_idx = edge_index[0]
    dst_idx = edge_index[1]
    nf_lo = node_feature[:, :DH]
    nf_hi = node_feature[:, DH:]
    fi_g, fj2_g = _gather_k(node_feature, nf_lo, nf_hi, dst_idx, src_idx)
    alpha3, m2d = _run_mlp(
        fi_g, fj2_g, geo_encoding, w0a, w0bl, w0bh, w0c, b0.reshape(1, D),
        W1.T, b1.reshape(1, D), W2.T, b2.reshape(1, D), W3.T, b3.reshape(1, D),
        att,
    )
    alpha = alpha3.reshape(E)
    m_arr = m2d.reshape(D)
    acc, den = _scatter_k(fj2_g, alpha, m_arr, dst_idx)
    return _run_combine(node_feature, acc, den)


# speedup vs baseline: 3.7522x; 1.1537x over previous
import functools

import jax
import jax.numpy as jnp
from jax import lax
from jax.experimental import pallas as pl
from jax.experimental.pallas import tpu as pltpu
from jax.experimental.pallas import tpu_sc as plsc

N = 10000
E = 320000
D = 128
G = 13
DH = D // 2

NC = 2
NS = 16
LANES = 16
NW = NC * NS
EPT = E // NW
EPT2 = E // NS
CH = 400
SUB = 80
NSUB = CH // SUB
ROWA = 624
TAIL = N - NS * ROWA

_MESH = plsc.VectorSubcoreMesh(
    core_axis_name="c", subcore_axis_name="s", num_cores=NC, num_subcores=NS
)
_SC_LINEAR = pltpu.CompilerParams(use_tc_tiling_on_sc=False)


@functools.partial(
    pl.kernel,
    out_type=(
        jax.ShapeDtypeStruct((E, D), jnp.float32),
        jax.ShapeDtypeStruct((E, D), jnp.float32),
    ),
    mesh=_MESH,
    scratch_types=[
        pltpu.VMEM((CH,), jnp.int32),
        pltpu.VMEM((CH,), jnp.int32),
        pltpu.VMEM((CH, D), jnp.float32),
        pltpu.VMEM((CH, D), jnp.float32),
        pltpu.SemaphoreType.DMA,
        pltpu.SemaphoreType.DMA,
    ],
)
def _gather_k(nf, dst, src, fi_out, fj_out, idx_i, idx_j, buf_i, buf_j,
              sem_i, sem_j):
    wid = lax.axis_index("s") * NC + lax.axis_index("c")
    base0 = wid * EPT

    def body(c, carry):
        base = base0 + c * CH
        pltpu.sync_copy(dst.at[pl.ds(base, CH)], idx_i)
        pltpu.sync_copy(src.at[pl.ds(base, CH)], idx_j)
        cp_i = pltpu.async_copy(nf.at[idx_i], buf_i, sem_i)
        cp_j = pltpu.async_copy(nf.at[idx_j], buf_j, sem_j)
        cp_i.wait()
        cp_j.wait()
        pltpu.sync_copy(buf_i, fi_out.at[pl.ds(base, CH)])
        pltpu.sync_copy(buf_j, fj_out.at[pl.ds(base, CH)])
        return carry

    lax.fori_loop(0, EPT // CH, body, 0)


BLK = 1280
NBLK = E // BLK


def _mlp_body(fi, fj, geo, w0a, w0b, w0c, b0, w1, b1, w2, b2, w3, b3,
              att, alpha_ref, m_ref, mscr):
    step = pl.program_id(0)

    @pl.when(step == 0)
    def _():
        mscr[0] = -jnp.inf

    bf = jnp.bfloat16

    def mm(a, w):
        return jnp.dot(a.astype(bf), w[...].astype(bf),
                       preferred_element_type=jnp.float32)

    h = (
        mm(fi[...], w0a) + mm(fj[...], w0b) + mm(geo[...], w0c) + b0[...]
    )
    h = jnp.maximum(h, 0.0)
    h = jnp.maximum(mm(h, w1) + b1[...], 0.0)
    h = jnp.maximum(mm(h, w2) + b2[...], 0.0)
    h = jnp.maximum(mm(h, w3) + b3[...], 0.0)
    y = h * att[...]
    y = jnp.where(y >= 0.0, y, 0.01 * y)
    a = jnp.sum(y, axis=1)
    alpha_ref[0, 0, :] = a
    mscr[0] = jnp.maximum(mscr[0], jnp.max(a))

    @pl.when(step == NBLK - 1)
    def _():
        m_ref[...] = jnp.full((1, D), mscr[0], jnp.float32)


def _run_mlp(fi_g, fj_g, geo, w0a, w0b, w0c, b0, w1, b1, w2, b2, w3, b3, att):
    full = lambda shape: pl.BlockSpec(shape, lambda i: (0,) * len(shape))
    alpha3, m2d = pl.pallas_call(
        _mlp_body,
        grid=(NBLK,),
        in_specs=[
            pl.BlockSpec((BLK, D), lambda i: (i, 0)),
            pl.BlockSpec((BLK, D), lambda i: (i, 0)),
            pl.BlockSpec((BLK, G), lambda i: (i, 0)),
            full((D, D)), full((D, D)), full((G, D)), full((1, D)),
            full((D, D)), full((1, D)),
            full((D, D)), full((1, D)),
            full((D, D)), full((1, D)),
            full((1, D)),
        ],
        out_specs=[
            pl.BlockSpec((1, 1, BLK), lambda i: (i, 0, 0)),
            pl.BlockSpec((1, D), lambda i: (0, 0)),
        ],
        out_shape=[
            jax.ShapeDtypeStruct((NBLK, 1, BLK), jnp.float32),
            jax.ShapeDtypeStruct((1, D), jnp.float32),
        ],
        scratch_shapes=[pltpu.SMEM((1,), jnp.float32)],
    )(fi_g, fj_g, geo, w0a, w0b, w0c, b0, w1, b1, w2, b2, w3, b3, att)
    return alpha3, m2d


@functools.partial(
    pl.kernel,
    out_type=(
        jax.ShapeDtypeStruct((NC, N, DH), jnp.float32),
        jax.ShapeDtypeStruct((NC, N, LANES), jnp.float32),
    ),
    mesh=_MESH,
    scratch_types=[
        pltpu.VMEM((CH, DH), jnp.float32),
        pltpu.VMEM((CH, LANES), jnp.float32),
        pltpu.VMEM((CH,), jnp.float32),
        pltpu.VMEM((CH,), jnp.int32),
        pltpu.VMEM((NSUB, SUB), jnp.int32),
        pltpu.VMEM((LANES,), jnp.float32),
        pltpu.SemaphoreType.DMA,
        pltpu.VMEM_SHARED((N, DH), jnp.float32),
        pltpu.VMEM_SHARED((N, LANES), jnp.float32),
    ],
    compiler_params=_SC_LINEAR,
)
def _scatter_k(nf_lo, nf_hi, alpha, m_arr, dst, src, acc_out, den_out,
               buf, dstage, aw, idxj, idxw, mv, sem, acc_sh, den_sh):
    cid = lax.axis_index("c")
    sid = lax.axis_index("s")
    base0 = sid * EPT2

    pltpu.sync_copy(m_arr.at[pl.ds(0, LANES)], mv)
    zv = jnp.zeros((LANES,), jnp.float32)

    def zrow(r, carry):
        for c in range(DH // LANES):
            buf[r, pl.ds(c * LANES, LANES)] = zv
        dstage[r, pl.ds(0, LANES)] = zv
        return carry

    lax.fori_loop(0, CH, zrow, 0)
    rows0 = sid * ROWA
    pltpu.sync_copy(buf, acc_sh.at[pl.ds(rows0, CH)])
    pltpu.sync_copy(buf.at[pl.ds(0, ROWA - CH)], acc_sh.at[pl.ds(rows0 + CH, ROWA - CH)])
    pltpu.sync_copy(dstage, den_sh.at[pl.ds(rows0, CH)])
    pltpu.sync_copy(dstage.at[pl.ds(0, ROWA - CH)], den_sh.at[pl.ds(rows0 + CH, ROWA - CH)])

    @pl.when(sid == NS - 1)
    def _():
        pltpu.sync_copy(buf.at[pl.ds(0, TAIL)], acc_sh.at[pl.ds(NS * ROWA, TAIL)])
        pltpu.sync_copy(dstage.at[pl.ds(0, TAIL)], den_sh.at[pl.ds(NS * ROWA, TAIL)])

    plsc.subcore_barrier()

    def chunk(c, carry):
        base = base0 + c * CH
        pltpu.sync_copy(src.at[pl.ds(base, CH)], idxj)
        pltpu.sync_copy(alpha.at[pl.ds(base, CH)], aw)
        for t in range(NSUB):
            pltpu.sync_copy(dst.at[pl.ds(base + t * SUB, SUB)], idxw.at[t])

        @pl.when(cid == 0)
        def _():
            pltpu.async_copy(nf_lo.at[idxj], buf, sem).wait()

        @pl.when(cid == 1)
        def _():
            pltpu.async_copy(nf_hi.at[idxj], buf, sem).wait()

        def grp(g, carry2):
            av = aw[pl.ds(g * LANES, LANES)]
            ex = jnp.exp(av - mv[...])
            for l in range(LANES):
                b = jnp.broadcast_to(ex[l], (LANES,))
                r = g * LANES + l
                dstage[r, pl.ds(0, LANES)] = b
                for c2 in range(DH // LANES):
                    sl = pl.ds(c2 * LANES, LANES)
                    buf[r, sl] = buf[r, sl] * b
            return carry2

        lax.fori_loop(0, CH // LANES, grp, 0)

        for t in range(NSUB):
            pltpu.sync_copy(buf.at[pl.ds(t * SUB, SUB)],
                            acc_sh.at[idxw.at[t]], add=True)

        @pl.when(cid == 0)
        def _():
            for t in range(NSUB):
                pltpu.sync_copy(dstage.at[pl.ds(t * SUB, SUB)],
                                den_sh.at[idxw.at[t]], add=True)

        return carry

    lax.fori_loop(0, EPT2 // CH, chunk, 0)
    plsc.subcore_barrier()

    pltpu.sync_copy(acc_sh.at[pl.ds(rows0, ROWA)],
                    acc_out.at[cid, pl.ds(rows0, ROWA)])
    pltpu.sync_copy(den_sh.at[pl.ds(rows0, ROWA)],
                    den_out.at[cid, pl.ds(rows0, ROWA)])

    @pl.when(sid == NS - 1)
    def _():
        pltpu.sync_copy(acc_sh.at[pl.ds(NS * ROWA, TAIL)],
                        acc_out.at[cid, pl.ds(NS * ROWA, TAIL)])
        pltpu.sync_copy(den_sh.at[pl.ds(NS * ROWA, TAIL)],
                        den_out.at[cid, pl.ds(NS * ROWA, TAIL)])


BN = 2000


def _combine_body(nf, acc, den, out):
    a = jnp.concatenate([acc[0], acc[1]], axis=-1)
    d = den[0, :, 0:1] + den[1, :, 0:1]
    d = jnp.where(d > 0.0, d, 1.0)
    out[...] = nf[...] + a / d


def _run_combine(node_feature, acc, den):
    return pl.pallas_call(
        _combine_body,
        grid=(N // BN,),
        in_specs=[
            pl.BlockSpec((BN, D), lambda i: (i, 0)),
            pl.BlockSpec((NC, BN, DH), lambda i: (0, i, 0)),
            pl.BlockSpec((NC, BN, LANES), lambda i: (0, i, 0)),
        ],
        out_specs=pl.BlockSpec((BN, D), lambda i: (i, 0)),
        out_shape=jax.ShapeDtypeStruct((N, D), jnp.float32),
    )(node_feature, acc, den)


def kernel(node_feature, geo_encoding, edge_index, is_source,
           W0, b0, W1, b1, W2, b2, W3, b3, att):
    del is_source
    w0t = W0.T
    w0a = w0t[:D]
    w0b = w0t[D:2 * D]
    w0c = w0t[2 * D:]
    src_idx = edge_index[0]
    dst_idx = edge_index[1]
    nf_lo = node_feature[:, :DH]
    nf_hi = node_feature[:, DH:]
    fi_g, fj_g = _gather_k(node_feature, dst_idx, src_idx)
    alpha3, m2d = _run_mlp(
        fi_g, fj_g, geo_encoding, w0a, w0b, w0c, b0.reshape(1, D),
        W1.T, b1.reshape(1, D), W2.T, b2.reshape(1, D), W3.T, b3.reshape(1, D),
        att,
    )
    alpha = alpha3.reshape(E)
    m_arr = m2d.reshape(D)
    acc, den = _scatter_k(nf_lo, nf_hi, alpha, m_arr, dst_idx, src_idx)
    return _run_combine(node_feature, acc, den)


# speedup vs baseline: 5.4850x; 1.4618x over previous
import functools

import jax
import jax.numpy as jnp
from jax import lax
from jax.experimental import pallas as pl
from jax.experimental.pallas import tpu as pltpu
from jax.experimental.pallas import tpu_sc as plsc

N = 10000
E = 320000
D = 128
G = 13
DH = D // 2

NC = 2
NS = 16
LANES = 16
NW = NC * NS
EPT = E // NW
EPT2 = E // NS
CH = 400
CH2 = 800
SUB = 100
NSUB = CH2 // SUB
ROWA = 624
TAIL = N - NS * ROWA

_MESH = plsc.VectorSubcoreMesh(
    core_axis_name="c", subcore_axis_name="s", num_cores=NC, num_subcores=NS
)
_SC_LINEAR = pltpu.CompilerParams(use_tc_tiling_on_sc=False)


@functools.partial(
    pl.kernel,
    out_type=(
        jax.ShapeDtypeStruct((E, D), jnp.float32),
        jax.ShapeDtypeStruct((E, D), jnp.float32),
    ),
    mesh=_MESH,
    scratch_types=[
        pltpu.VMEM((CH,), jnp.int32),
        pltpu.VMEM((CH,), jnp.int32),
        pltpu.VMEM((CH, D), jnp.float32),
        pltpu.VMEM((CH, D), jnp.float32),
        pltpu.SemaphoreType.DMA,
        pltpu.SemaphoreType.DMA,
        pltpu.SemaphoreType.DMA,
        pltpu.SemaphoreType.DMA,
    ],
)
def _gather_k(nf, dst, src, fi_out, fj_out, idx_i, idx_j, buf_i, buf_j,
              sem_ix, sem_i, sem_j, sem_wr):
    wid = lax.axis_index("s") * NC + lax.axis_index("c")
    base0 = wid * EPT

    def body(c, carry):
        base = base0 + c * CH

        @pl.when(c > 0)
        def _():
            pltpu.make_async_copy(buf_i, fi_out.at[pl.ds(base, CH)], sem_wr).wait()
            pltpu.make_async_copy(buf_j, fj_out.at[pl.ds(base, CH)], sem_wr).wait()

        ci = pltpu.async_copy(dst.at[pl.ds(base, CH)], idx_i, sem_ix)
        cj = pltpu.async_copy(src.at[pl.ds(base, CH)], idx_j, sem_ix)
        ci.wait()
        cj.wait()
        gi = pltpu.async_copy(nf.at[idx_i], buf_i, sem_i)
        gj = pltpu.async_copy(nf.at[idx_j], buf_j, sem_j)
        gi.wait()
        gj.wait()
        pltpu.async_copy(buf_i, fi_out.at[pl.ds(base, CH)], sem_wr)
        pltpu.async_copy(buf_j, fj_out.at[pl.ds(base, CH)], sem_wr)
        return carry

    lax.fori_loop(0, EPT // CH, body, 0)
    pltpu.make_async_copy(buf_i, fi_out.at[pl.ds(base0, CH)], sem_wr).wait()
    pltpu.make_async_copy(buf_j, fj_out.at[pl.ds(base0, CH)], sem_wr).wait()


BLK = 2560
NBLK = E // BLK


def _mlp_body(fi, fj, geot, w0a, w0b, w0c, b0, w1, b1, w2, b2, w3, b3,
              att, ones, alpha_ref, m_ref, mscr):
    step = pl.program_id(0)

    @pl.when(step == 0)
    def _():
        mscr[0] = -jnp.inf

    bf = jnp.bfloat16
    f32 = jnp.float32

    def mm(a, w):
        return jnp.dot(a.astype(bf), w[...], preferred_element_type=f32)

    hg = lax.dot_general(geot[...].astype(bf), w0c[...],
                         (((0,), (0,)), ((), ())),
                         preferred_element_type=f32)
    h = mm(fi[...], w0a) + mm(fj[...], w0b) + hg + b0[...]
    h = jnp.maximum(h, 0.0)
    h = jnp.maximum(mm(h, w1) + b1[...], 0.0)
    h = jnp.maximum(mm(h, w2) + b2[...], 0.0)
    h = jnp.maximum(mm(h, w3) + b3[...], 0.0)
    y = h * att[...]
    y = jnp.where(y >= 0.0, y, 0.01 * y)
    z = mm(y, ones)
    zc = z[:, 0:1]
    alpha_ref[0, :, :] = zc
    mscr[0] = jnp.maximum(mscr[0], jnp.max(zc))

    @pl.when(step == NBLK - 1)
    def _():
        m_ref[...] = jnp.full((1, D), mscr[0], jnp.float32)


def _run_mlp(fi_g, fj_g, geot, w0a, w0b, w0c, b0, w1, b1, w2, b2, w3, b3,
             att, ones):
    full = lambda shape: pl.BlockSpec(shape, lambda i: (0,) * len(shape))
    alpha3, m2d = pl.pallas_call(
        _mlp_body,
        grid=(NBLK,),
        in_specs=[
            pl.BlockSpec((BLK, D), lambda i: (i, 0)),
            pl.BlockSpec((BLK, D), lambda i: (i, 0)),
            pl.BlockSpec((G, BLK), lambda i: (0, i)),
            full((D, D)), full((D, D)), full((G, D)), full((1, D)),
            full((D, D)), full((1, D)),
            full((D, D)), full((1, D)),
            full((D, D)), full((1, D)),
            full((1, D)), full((D, D)),
        ],
        out_specs=[
            pl.BlockSpec((1, BLK, 1), lambda i: (i, 0, 0)),
            pl.BlockSpec((1, D), lambda i: (0, 0)),
        ],
        out_shape=[
            jax.ShapeDtypeStruct((NBLK, BLK, 1), jnp.float32),
            jax.ShapeDtypeStruct((1, D), jnp.float32),
        ],
        scratch_shapes=[pltpu.SMEM((1,), jnp.float32)],
    )(fi_g, fj_g, geot, w0a, w0b, w0c, b0, w1, b1, w2, b2, w3, b3, att, ones)
    return alpha3, m2d


@functools.partial(
    pl.kernel,
    out_type=(
        jax.ShapeDtypeStruct((NC, N, DH), jnp.float32),
        jax.ShapeDtypeStruct((NC, N, LANES), jnp.float32),
    ),
    mesh=_MESH,
    scratch_types=[
        pltpu.VMEM((CH2, DH), jnp.float32),
        pltpu.VMEM((CH2, LANES), jnp.float32),
        pltpu.VMEM((CH2,), jnp.float32),
        pltpu.VMEM((CH2,), jnp.int32),
        pltpu.VMEM((NSUB, SUB), jnp.int32),
        pltpu.VMEM((LANES,), jnp.float32),
        pltpu.SemaphoreType.DMA,
        pltpu.SemaphoreType.DMA,
        pltpu.SemaphoreType.DMA,
        pltpu.SemaphoreType.DMA,
        pltpu.VMEM_SHARED((N, DH), jnp.float32),
        pltpu.VMEM_SHARED((N, LANES), jnp.float32),
    ],
    compiler_params=_SC_LINEAR,
)
def _scatter_k(nf_lo, nf_hi, alpha, m_arr, dst2d, src, acc_out, den_out,
               buf, dstage, aw, idxj, idxw, mv, sem_in, sem_rows, sem_acc,
               sem_den, acc_sh, den_sh):
    cid = lax.axis_index("c")
    sid = lax.axis_index("s")
    base0 = sid * EPT2

    pltpu.sync_copy(m_arr.at[pl.ds(0, LANES)], mv)
    zv = jnp.zeros((LANES,), jnp.float32)

    def zrow(r, carry):
        for c in range(DH // LANES):
            buf[r, pl.ds(c * LANES, LANES)] = zv
        dstage[r, pl.ds(0, LANES)] = zv
        return carry

    lax.fori_loop(0, ROWA, zrow, 0)
    rows0 = sid * ROWA
    pltpu.sync_copy(buf.at[pl.ds(0, ROWA)], acc_sh.at[pl.ds(rows0, ROWA)])
    pltpu.sync_copy(dstage.at[pl.ds(0, ROWA)], den_sh.at[pl.ds(rows0, ROWA)])

    @pl.when(sid == NS - 1)
    def _():
        pltpu.sync_copy(buf.at[pl.ds(0, TAIL)], acc_sh.at[pl.ds(NS * ROWA, TAIL)])
        pltpu.sync_copy(dstage.at[pl.ds(0, TAIL)], den_sh.at[pl.ds(NS * ROWA, TAIL)])

    plsc.subcore_barrier()

    def chunk(c, carry):
        base = base0 + c * CH2
        rw0 = sid * (EPT2 // SUB) + c * NSUB
        c1 = pltpu.async_copy(src.at[pl.ds(base, CH2)], idxj, sem_in)
        c2 = pltpu.async_copy(alpha.at[pl.ds(base, CH2)], aw, sem_in)
        c3 = pltpu.async_copy(dst2d.at[pl.ds(rw0, NSUB)], idxw, sem_in)
        c1.wait()
        c2.wait()
        c3.wait()

        @pl.when(cid == 0)
        def _():
            pltpu.async_copy(nf_lo.at[idxj], buf, sem_rows).wait()

        @pl.when(cid == 1)
        def _():
            pltpu.async_copy(nf_hi.at[idxj], buf, sem_rows).wait()

        def grp(g, carry2):
            av = aw[pl.ds(g * LANES, LANES)]
            ex = jnp.exp(av - mv[...])
            for l in range(LANES):
                b = jnp.broadcast_to(ex[l], (LANES,))
                r = g * LANES + l
                dstage[r, pl.ds(0, LANES)] = b
                for c2_ in range(DH // LANES):
                    sl = pl.ds(c2_ * LANES, LANES)
                    buf[r, sl] = buf[r, sl] * b
            return carry2

        lax.fori_loop(0, CH2 // LANES, grp, 0)

        for t in range(NSUB):
            pltpu.async_copy(buf.at[pl.ds(t * SUB, SUB)],
                             acc_sh.at[idxw.at[t]], sem_acc, add=True)

        @pl.when(cid == 0)
        def _():
            for t in range(NSUB):
                pltpu.async_copy(dstage.at[pl.ds(t * SUB, SUB)],
                                 den_sh.at[idxw.at[t]], sem_den, add=True)
            for t in range(NSUB):
                pltpu.make_async_copy(dstage.at[pl.ds(t * SUB, SUB)],
                                      den_sh.at[idxw.at[t]], sem_den).wait()

        for t in range(NSUB):
            pltpu.make_async_copy(buf.at[pl.ds(t * SUB, SUB)],
                                  acc_sh.at[idxw.at[t]], sem_acc).wait()
        return carry

    lax.fori_loop(0, EPT2 // CH2, chunk, 0)
    plsc.subcore_barrier()

    pltpu.sync_copy(acc_sh.at[pl.ds(rows0, ROWA)],
                    acc_out.at[cid, pl.ds(rows0, ROWA)])
    pltpu.sync_copy(den_sh.at[pl.ds(rows0, ROWA)],
                    den_out.at[cid, pl.ds(rows0, ROWA)])

    @pl.when(sid == NS - 1)
    def _():
        pltpu.sync_copy(acc_sh.at[pl.ds(NS * ROWA, TAIL)],
                        acc_out.at[cid, pl.ds(NS * ROWA, TAIL)])
        pltpu.sync_copy(den_sh.at[pl.ds(NS * ROWA, TAIL)],
                        den_out.at[cid, pl.ds(NS * ROWA, TAIL)])


BN = 2000


def _combine_body(nf, acc, den, out):
    a = jnp.concatenate([acc[0], acc[1]], axis=-1)
    d = den[0, :, 0:1] + den[1, :, 0:1]
    d = jnp.where(d > 0.0, d, 1.0)
    out[...] = nf[...] + a / d


def _run_combine(node_feature, acc, den):
    return pl.pallas_call(
        _combine_body,
        grid=(N // BN,),
        in_specs=[
            pl.BlockSpec((BN, D), lambda i: (i, 0)),
            pl.BlockSpec((NC, BN, DH), lambda i: (0, i, 0)),
            pl.BlockSpec((NC, BN, LANES), lambda i: (0, i, 0)),
        ],
        out_specs=pl.BlockSpec((BN, D), lambda i: (i, 0)),
        out_shape=jax.ShapeDtypeStruct((N, D), jnp.float32),
    )(node_feature, acc, den)


def kernel(node_feature, geo_encoding, edge_index, is_source,
           W0, b0, W1, b1, W2, b2, W3, b3, att):
    del is_source
    bf = jnp.bfloat16
    w0t = W0.T.astype(bf)
    w0a = w0t[:D]
    w0b = w0t[D:2 * D]
    w0c = w0t[2 * D:]
    ones = jnp.ones((D, D), bf)
    src_idx = edge_index[0]
    dst_idx = edge_index[1]
    dst2d = dst_idx.reshape(E // SUB, SUB)
    geot = geo_encoding.T
    nf_lo = node_feature[:, :DH]
    nf_hi = node_feature[:, DH:]
    fi_g, fj_g = _gather_k(node_feature, dst_idx, src_idx)
    alpha3, m2d = _run_mlp(
        fi_g, fj_g, geot, w0a, w0b, w0c, b0.reshape(1, D),
        W1.T.astype(bf), b1.reshape(1, D), W2.T.astype(bf), b2.reshape(1, D),
        W3.T.astype(bf), b3.reshape(1, D), att, ones,
    )
    alpha = alpha3.reshape(E)
    m_arr = m2d.reshape(D)
    acc, den = _scatter_k(nf_lo, nf_hi, alpha, m_arr, dst2d, src_idx)
    return _run_combine(node_feature, acc, den)


# speedup vs baseline: 5.8503x; 1.0666x over previous
import functools

import jax
import jax.numpy as jnp
from jax import lax
from jax.experimental import pallas as pl
from jax.experimental.pallas import tpu as pltpu
from jax.experimental.pallas import tpu_sc as plsc

N = 10000
E = 320000
D = 128
G = 13
DH = D // 2

NC = 2
NS = 16
LANES = 16
NW = NC * NS
EPT = E // NW
EPT2 = E // NS
CH = 400
CH2 = 400
SUB = 100
NSUB = CH2 // SUB
NIT = EPT2 // CH2 // 2
ROWA = 624
TAIL = N - NS * ROWA

_MESH = plsc.VectorSubcoreMesh(
    core_axis_name="c", subcore_axis_name="s", num_cores=NC, num_subcores=NS
)
_SC_LINEAR = pltpu.CompilerParams(use_tc_tiling_on_sc=False)


@functools.partial(
    pl.kernel,
    out_type=(
        jax.ShapeDtypeStruct((E, D), jnp.float32),
        jax.ShapeDtypeStruct((E, D), jnp.float32),
    ),
    mesh=_MESH,
    scratch_types=[
        pltpu.VMEM((CH,), jnp.int32),
        pltpu.VMEM((CH,), jnp.int32),
        pltpu.VMEM((CH, D), jnp.float32),
        pltpu.VMEM((CH, D), jnp.float32),
        pltpu.SemaphoreType.DMA,
        pltpu.SemaphoreType.DMA,
        pltpu.SemaphoreType.DMA,
        pltpu.SemaphoreType.DMA,
    ],
)
def _gather_k(nf, dst, src, fi_out, fj_out, idx_i, idx_j, buf_i, buf_j,
              sem_ix, sem_i, sem_j, sem_wr):
    wid = lax.axis_index("s") * NC + lax.axis_index("c")
    base0 = wid * EPT

    def body(c, carry):
        base = base0 + c * CH

        @pl.when(c > 0)
        def _():
            pltpu.make_async_copy(buf_i, fi_out.at[pl.ds(base, CH)], sem_wr).wait()
            pltpu.make_async_copy(buf_j, fj_out.at[pl.ds(base, CH)], sem_wr).wait()

        ci = pltpu.async_copy(dst.at[pl.ds(base, CH)], idx_i, sem_ix)
        cj = pltpu.async_copy(src.at[pl.ds(base, CH)], idx_j, sem_ix)
        ci.wait()
        cj.wait()
        gi = pltpu.async_copy(nf.at[idx_i], buf_i, sem_i)
        gj = pltpu.async_copy(nf.at[idx_j], buf_j, sem_j)
        gi.wait()
        gj.wait()
        pltpu.async_copy(buf_i, fi_out.at[pl.ds(base, CH)], sem_wr)
        pltpu.async_copy(buf_j, fj_out.at[pl.ds(base, CH)], sem_wr)
        return carry

    lax.fori_loop(0, EPT // CH, body, 0)
    pltpu.make_async_copy(buf_i, fi_out.at[pl.ds(base0, CH)], sem_wr).wait()
    pltpu.make_async_copy(buf_j, fj_out.at[pl.ds(base0, CH)], sem_wr).wait()


BLK = 2560
NBLK = E // BLK


def _mlp_body(fi, fj, geot, w0a, w0b, w0c, b0, w1, b1, w2, b2, w3, b3,
              att, ones, alpha_ref, m_ref, mscr):
    step = pl.program_id(0)

    @pl.when(step == 0)
    def _():
        mscr[0] = -jnp.inf

    bf = jnp.bfloat16
    f32 = jnp.float32

    def mm(a, w):
        return jnp.dot(a.astype(bf), w[...], preferred_element_type=f32)

    hg = lax.dot_general(geot[...].astype(bf), w0c[...],
                         (((0,), (0,)), ((), ())),
                         preferred_element_type=f32)
    h = mm(fi[...], w0a) + mm(fj[...], w0b) + hg + b0[...]
    h = jnp.maximum(h, 0.0)
    h = jnp.maximum(mm(h, w1) + b1[...], 0.0)
    h = jnp.maximum(mm(h, w2) + b2[...], 0.0)
    h = jnp.maximum(mm(h, w3) + b3[...], 0.0)
    y = h * att[...]
    y = jnp.where(y >= 0.0, y, 0.01 * y)
    zt = lax.dot_general(ones[...], y.astype(bf), (((0,), (1,)), ((), ())),
                         preferred_element_type=f32)
    alpha_ref[0, 0, :] = zt[0]
    mscr[0] = jnp.maximum(mscr[0], jnp.max(zt[0]))

    @pl.when(step == NBLK - 1)
    def _():
        m_ref[...] = jnp.full((1, D), mscr[0], jnp.float32)


def _run_mlp(fi_g, fj_g, geot, w0a, w0b, w0c, b0, w1, b1, w2, b2, w3, b3,
             att, ones):
    full = lambda shape: pl.BlockSpec(shape, lambda i: (0,) * len(shape))
    alpha3, m2d = pl.pallas_call(
        _mlp_body,
        grid=(NBLK,),
        in_specs=[
            pl.BlockSpec((BLK, D), lambda i: (i, 0)),
            pl.BlockSpec((BLK, D), lambda i: (i, 0)),
            pl.BlockSpec((G, BLK), lambda i: (0, i)),
            full((D, D)), full((D, D)), full((G, D)), full((1, D)),
            full((D, D)), full((1, D)),
            full((D, D)), full((1, D)),
            full((D, D)), full((1, D)),
            full((1, D)), full((D, D)),
        ],
        out_specs=[
            pl.BlockSpec((1, 1, BLK), lambda i: (i, 0, 0)),
            pl.BlockSpec((1, D), lambda i: (0, 0)),
        ],
        out_shape=[
            jax.ShapeDtypeStruct((NBLK, 1, BLK), jnp.float32),
            jax.ShapeDtypeStruct((1, D), jnp.float32),
        ],
        scratch_shapes=[pltpu.SMEM((1,), jnp.float32)],
    )(fi_g, fj_g, geot, w0a, w0b, w0c, b0, w1, b1, w2, b2, w3, b3, att, ones)
    return alpha3, m2d


@functools.partial(
    pl.kernel,
    out_type=(
        jax.ShapeDtypeStruct((NC, N, DH), jnp.float32),
        jax.ShapeDtypeStruct((NC, N, LANES), jnp.float32),
    ),
    mesh=_MESH,
    scratch_types=[
        pltpu.VMEM((CH2, DH), jnp.float32),
        pltpu.VMEM((CH2, DH), jnp.float32),
        pltpu.VMEM((CH2, LANES), jnp.float32),
        pltpu.VMEM((CH2, LANES), jnp.float32),
        pltpu.VMEM((CH2,), jnp.float32),
        pltpu.VMEM((CH2,), jnp.float32),
        pltpu.VMEM((CH2,), jnp.int32),
        pltpu.VMEM((CH2,), jnp.int32),
        pltpu.VMEM((NSUB, SUB), jnp.int32),
        pltpu.VMEM((NSUB, SUB), jnp.int32),
        pltpu.VMEM((LANES,), jnp.float32),
        pltpu.SemaphoreType.DMA,
        pltpu.SemaphoreType.DMA,
        pltpu.SemaphoreType.DMA,
        pltpu.SemaphoreType.DMA,
        pltpu.VMEM_SHARED((N, DH), jnp.float32),
        pltpu.VMEM_SHARED((N, LANES), jnp.float32),
    ],
    compiler_params=_SC_LINEAR,
)
def _scatter_k(nf_lo, nf_hi, alpha, m_arr, dst2d, src, acc_out, den_out,
               buf_a, buf_b, ds_a, ds_b, aw_a, aw_b, ij_a, ij_b, iw_a, iw_b,
               mv, sem_sm, sem_rows, sem_acc, sem_den, acc_sh, den_sh):
    cid = lax.axis_index("c")
    sid = lax.axis_index("s")
    base0 = sid * EPT2

    slot_a = (buf_a, ds_a, aw_a, ij_a, iw_a)
    slot_b = (buf_b, ds_b, aw_b, ij_b, iw_b)

    pltpu.sync_copy(m_arr.at[pl.ds(0, LANES)], mv)
    zv = jnp.zeros((LANES,), jnp.float32)

    def zrow(r, carry):
        for c in range(DH // LANES):
            buf_a[r, pl.ds(c * LANES, LANES)] = zv
            buf_b[r, pl.ds(c * LANES, LANES)] = zv
        ds_a[r, pl.ds(0, LANES)] = zv
        ds_b[r, pl.ds(0, LANES)] = zv
        return carry

    lax.fori_loop(0, CH2, zrow, 0)
    rows0 = sid * ROWA
    pltpu.sync_copy(buf_a, acc_sh.at[pl.ds(rows0, CH2)])
    pltpu.sync_copy(buf_a.at[pl.ds(0, ROWA - CH2)],
                    acc_sh.at[pl.ds(rows0 + CH2, ROWA - CH2)])
    pltpu.sync_copy(ds_a, den_sh.at[pl.ds(rows0, CH2)])
    pltpu.sync_copy(ds_a.at[pl.ds(0, ROWA - CH2)],
                    den_sh.at[pl.ds(rows0 + CH2, ROWA - CH2)])

    @pl.when(sid == NS - 1)
    def _():
        pltpu.sync_copy(buf_a.at[pl.ds(0, TAIL)], acc_sh.at[pl.ds(NS * ROWA, TAIL)])
        pltpu.sync_copy(ds_a.at[pl.ds(0, TAIL)], den_sh.at[pl.ds(NS * ROWA, TAIL)])

    plsc.subcore_barrier()

    def issue_smalls(slot, c):
        _, _, aw_, ij_, iw_ = slot
        base = base0 + c * CH2
        rw0 = sid * (EPT2 // SUB) + c * NSUB
        pltpu.async_copy(src.at[pl.ds(base, CH2)], ij_, sem_sm)
        pltpu.async_copy(alpha.at[pl.ds(base, CH2)], aw_, sem_sm)
        pltpu.async_copy(dst2d.at[pl.ds(rw0, NSUB)], iw_, sem_sm)

    def wait_smalls(slot):
        _, _, aw_, ij_, iw_ = slot
        pltpu.make_async_copy(src.at[pl.ds(base0, CH2)], ij_, sem_sm).wait()
        pltpu.make_async_copy(alpha.at[pl.ds(base0, CH2)], aw_, sem_sm).wait()
        pltpu.make_async_copy(dst2d.at[pl.ds(0, NSUB)], iw_, sem_sm).wait()

    def issue_rows(slot):
        buf_, _, _, ij_, _ = slot

        @pl.when(cid == 0)
        def _():
            pltpu.async_copy(nf_lo.at[ij_], buf_, sem_rows)

        @pl.when(cid == 1)
        def _():
            pltpu.async_copy(nf_hi.at[ij_], buf_, sem_rows)

    def wait_rows(slot):
        buf_, _, _, ij_, _ = slot
        pltpu.make_async_copy(nf_lo.at[ij_], buf_, sem_rows).wait()

    def compute(slot):
        buf_, ds_, aw_, _, _ = slot

        def grp(g, carry2):
            av = aw_[pl.ds(g * LANES, LANES)]
            ex = jnp.exp(av - mv[...])
            for l in range(LANES):
                b = jnp.broadcast_to(ex[l], (LANES,))
                r = g * LANES + l
                ds_[r, pl.ds(0, LANES)] = b
                for c2_ in range(DH // LANES):
                    sl = pl.ds(c2_ * LANES, LANES)
                    buf_[r, sl] = buf_[r, sl] * b
            return carry2

        lax.fori_loop(0, CH2 // LANES, grp, 0)

    def issue_scatter(slot):
        buf_, ds_, _, _, iw_ = slot
        for t in range(NSUB):
            pltpu.async_copy(buf_.at[pl.ds(t * SUB, SUB)],
                             acc_sh.at[iw_.at[t]], sem_acc, add=True)

        @pl.when(cid == 0)
        def _():
            for t in range(NSUB):
                pltpu.async_copy(ds_.at[pl.ds(t * SUB, SUB)],
                                 den_sh.at[iw_.at[t]], sem_den, add=True)

    def wait_scatter(slot):
        buf_, ds_, _, _, iw_ = slot
        for t in range(NSUB):
            pltpu.make_async_copy(buf_.at[pl.ds(t * SUB, SUB)],
                                  acc_sh.at[iw_.at[t]], sem_acc).wait()

        @pl.when(cid == 0)
        def _():
            for t in range(NSUB):
                pltpu.make_async_copy(ds_.at[pl.ds(t * SUB, SUB)],
                                      den_sh.at[iw_.at[t]], sem_den).wait()

    issue_smalls(slot_a, 0)
    wait_smalls(slot_a)
    issue_rows(slot_a)

    def body(it, carry):
        a = 2 * it
        wait_rows(slot_a)

        @pl.when(it > 0)
        def _():
            wait_scatter(slot_b)

        issue_smalls(slot_b, a + 1)
        compute(slot_a)
        wait_smalls(slot_b)
        issue_rows(slot_b)
        issue_scatter(slot_a)
        wait_rows(slot_b)
        wait_scatter(slot_a)

        @pl.when(it < NIT - 1)
        def _():
            issue_smalls(slot_a, a + 2)

        compute(slot_b)

        @pl.when(it < NIT - 1)
        def _():
            wait_smalls(slot_a)
            issue_rows(slot_a)

        issue_scatter(slot_b)
        return carry

    lax.fori_loop(0, NIT, body, 0)
    wait_scatter(slot_b)
    plsc.subcore_barrier()

    pltpu.sync_copy(acc_sh.at[pl.ds(rows0, ROWA)],
                    acc_out.at[cid, pl.ds(rows0, ROWA)])
    pltpu.sync_copy(den_sh.at[pl.ds(rows0, ROWA)],
                    den_out.at[cid, pl.ds(rows0, ROWA)])

    @pl.when(sid == NS - 1)
    def _():
        pltpu.sync_copy(acc_sh.at[pl.ds(NS * ROWA, TAIL)],
                        acc_out.at[cid, pl.ds(NS * ROWA, TAIL)])
        pltpu.sync_copy(den_sh.at[pl.ds(NS * ROWA, TAIL)],
                        den_out.at[cid, pl.ds(NS * ROWA, TAIL)])


BN = 2000


def _combine_body(nf, acc, den, out):
    a = jnp.concatenate([acc[0], acc[1]], axis=-1)
    d = den[0, :, 0:1] + den[1, :, 0:1]
    d = jnp.where(d > 0.0, d, 1.0)
    out[...] = nf[...] + a / d


def _run_combine(node_feature, acc, den):
    return pl.pallas_call(
        _combine_body,
        grid=(N // BN,),
        in_specs=[
            pl.BlockSpec((BN, D), lambda i: (i, 0)),
            pl.BlockSpec((NC, BN, DH), lambda i: (0, i, 0)),
            pl.BlockSpec((NC, BN, LANES), lambda i: (0, i, 0)),
        ],
        out_specs=pl.BlockSpec((BN, D), lambda i: (i, 0)),
        out_shape=jax.ShapeDtypeStruct((N, D), jnp.float32),
    )(node_feature, acc, den)


def kernel(node_feature, geo_encoding, edge_index, is_source,
           W0, b0, W1, b1, W2, b2, W3, b3, att):
    del is_source
    bf = jnp.bfloat16
    w0t = W0.T.astype(bf)
    w0a = w0t[:D]
    w0b = w0t[D:2 * D]
    w0c = w0t[2 * D:]
    ones = jnp.ones((D, D), bf)
    src_idx = edge_index[0]
    dst_idx = edge_index[1]
    dst2d = dst_idx.reshape(E // SUB, SUB)
    geot = geo_encoding.T
    nf_lo = node_feature[:, :DH]
    nf_hi = node_feature[:, DH:]
    fi_g, fj_g = _gather_k(node_feature, dst_idx, src_idx)
    alpha3, m2d = _run_mlp(
        fi_g, fj_g, geot, w0a, w0b, w0c, b0.reshape(1, D),
        W1.T.astype(bf), b1.reshape(1, D), W2.T.astype(bf), b2.reshape(1, D),
        W3.T.astype(bf), b3.reshape(1, D), att, ones,
    )
    alpha = alpha3.reshape(E)
    m_arr = m2d.reshape(D)
    acc, den = _scatter_k(nf_lo, nf_hi, alpha, m_arr, dst2d, src_idx)
    return _run_combine(node_feature, acc, den)


# speedup vs baseline: 5.9793x; 1.0220x over previous
import functools

import jax
import jax.numpy as jnp
from jax import lax
from jax.experimental import pallas as pl
from jax.experimental.pallas import tpu as pltpu
from jax.experimental.pallas import tpu_sc as plsc

N = 10000
E = 320000
D = 128
G = 13
DH = D // 2

NC = 2
NS = 16
LANES = 16
NW = NC * NS
EPT = E // NW
EPT2 = E // NS
CH = 400
CH2 = 400
SUB = 100
NSUB = CH2 // SUB
NIT = EPT2 // CH2 // 2
ROWA = 624
TAIL = N - NS * ROWA

_MESH = plsc.VectorSubcoreMesh(
    core_axis_name="c", subcore_axis_name="s", num_cores=NC, num_subcores=NS
)
_SC_LINEAR = pltpu.CompilerParams(use_tc_tiling_on_sc=False,
                                  needs_layout_passes=False)


@functools.partial(
    pl.kernel,
    out_type=(
        jax.ShapeDtypeStruct((E, D), jnp.float32),
        jax.ShapeDtypeStruct((E, D), jnp.float32),
    ),
    mesh=_MESH,
    scratch_types=[
        pltpu.VMEM((CH,), jnp.int32),
        pltpu.VMEM((CH,), jnp.int32),
        pltpu.VMEM((CH, D), jnp.float32),
        pltpu.VMEM((CH, D), jnp.float32),
        pltpu.SemaphoreType.DMA,
        pltpu.SemaphoreType.DMA,
        pltpu.SemaphoreType.DMA,
        pltpu.SemaphoreType.DMA,
    ],
)
def _gather_k(nf, dst, src, fi_out, fj_out, idx_i, idx_j, buf_i, buf_j,
              sem_ix, sem_i, sem_j, sem_wr):
    wid = lax.axis_index("s") * NC + lax.axis_index("c")
    base0 = wid * EPT

    def body(c, carry):
        base = base0 + c * CH

        @pl.when(c > 0)
        def _():
            pltpu.make_async_copy(buf_i, fi_out.at[pl.ds(base, CH)], sem_wr).wait()
            pltpu.make_async_copy(buf_j, fj_out.at[pl.ds(base, CH)], sem_wr).wait()

        ci = pltpu.async_copy(dst.at[pl.ds(base, CH)], idx_i, sem_ix)
        cj = pltpu.async_copy(src.at[pl.ds(base, CH)], idx_j, sem_ix)
        ci.wait()
        cj.wait()
        gi = pltpu.async_copy(nf.at[idx_i], buf_i, sem_i)
        gj = pltpu.async_copy(nf.at[idx_j], buf_j, sem_j)
        gi.wait()
        gj.wait()
        pltpu.async_copy(buf_i, fi_out.at[pl.ds(base, CH)], sem_wr)
        pltpu.async_copy(buf_j, fj_out.at[pl.ds(base, CH)], sem_wr)
        return carry

    lax.fori_loop(0, EPT // CH, body, 0)
    pltpu.make_async_copy(buf_i, fi_out.at[pl.ds(base0, CH)], sem_wr).wait()
    pltpu.make_async_copy(buf_j, fj_out.at[pl.ds(base0, CH)], sem_wr).wait()


BLK = 2560
NBLK = E // BLK


def _mlp_body(fi, fj, geot, w0a, w0b, w0c, b0, w1, b1, w2, b2, w3, b3,
              att, ones, alpha_ref, m_ref, mscr):
    step = pl.program_id(0)

    @pl.when(step == 0)
    def _():
        mscr[0] = -jnp.inf

    bf = jnp.bfloat16
    f32 = jnp.float32

    def mm(a, w):
        return jnp.dot(a.astype(bf), w[...], preferred_element_type=f32)

    hg = lax.dot_general(geot[...].astype(bf), w0c[...],
                         (((0,), (0,)), ((), ())),
                         preferred_element_type=f32)
    h = mm(fi[...], w0a) + mm(fj[...], w0b) + hg + b0[...]
    h = jnp.maximum(h, 0.0)
    h = jnp.maximum(mm(h, w1) + b1[...], 0.0)
    h = jnp.maximum(mm(h, w2) + b2[...], 0.0)
    h = jnp.maximum(mm(h, w3) + b3[...], 0.0)
    y = h * att[...]
    y = jnp.where(y >= 0.0, y, 0.01 * y)
    zt = lax.dot_general(ones[...], y.astype(bf), (((0,), (1,)), ((), ())),
                         preferred_element_type=f32)
    alpha_ref[0, 0, :] = zt[0]
    mscr[0] = jnp.maximum(mscr[0], jnp.max(zt[0]))

    @pl.when(step == NBLK - 1)
    def _():
        m_ref[...] = jnp.full((1, D), mscr[0], jnp.float32)


def _run_mlp(fi_g, fj_g, geot, w0a, w0b, w0c, b0, w1, b1, w2, b2, w3, b3,
             att, ones):
    full = lambda shape: pl.BlockSpec(shape, lambda i: (0,) * len(shape))
    alpha3, m2d = pl.pallas_call(
        _mlp_body,
        grid=(NBLK,),
        in_specs=[
            pl.BlockSpec((BLK, D), lambda i: (i, 0)),
            pl.BlockSpec((BLK, D), lambda i: (i, 0)),
            pl.BlockSpec((G, BLK), lambda i: (0, i)),
            full((D, D)), full((D, D)), full((G, D)), full((1, D)),
            full((D, D)), full((1, D)),
            full((D, D)), full((1, D)),
            full((D, D)), full((1, D)),
            full((1, D)), full((D, D)),
        ],
        out_specs=[
            pl.BlockSpec((1, 1, BLK), lambda i: (i, 0, 0)),
            pl.BlockSpec((1, D), lambda i: (0, 0)),
        ],
        out_shape=[
            jax.ShapeDtypeStruct((NBLK, 1, BLK), jnp.float32),
            jax.ShapeDtypeStruct((1, D), jnp.float32),
        ],
        scratch_shapes=[pltpu.SMEM((1,), jnp.float32)],
    )(fi_g, fj_g, geot, w0a, w0b, w0c, b0, w1, b1, w2, b2, w3, b3, att, ones)
    return alpha3, m2d


@functools.partial(
    pl.kernel,
    out_type=(
        jax.ShapeDtypeStruct((NC, N, DH), jnp.float32),
        jax.ShapeDtypeStruct((NS, N), jnp.float32),
    ),
    mesh=_MESH,
    scratch_types=[
        pltpu.VMEM((CH2, DH), jnp.float32),
        pltpu.VMEM((CH2, DH), jnp.float32),
        pltpu.VMEM((CH2,), jnp.float32),
        pltpu.VMEM((CH2,), jnp.float32),
        pltpu.VMEM((CH2,), jnp.int32),
        pltpu.VMEM((CH2,), jnp.int32),
        pltpu.VMEM((CH2,), jnp.int32),
        pltpu.VMEM((CH2,), jnp.int32),
        pltpu.VMEM((NSUB, SUB), jnp.int32),
        pltpu.VMEM((NSUB, SUB), jnp.int32),
        pltpu.VMEM((N,), jnp.float32),
        pltpu.VMEM((LANES,), jnp.float32),
        pltpu.SemaphoreType.DMA,
        pltpu.SemaphoreType.DMA,
        pltpu.SemaphoreType.DMA,
        pltpu.VMEM_SHARED((N, DH), jnp.float32),
    ],
    compiler_params=_SC_LINEAR,
)
def _scatter_k(nf_lo, nf_hi, alpha, m_arr, dst2d, src, dst, acc_out, den_out,
               buf_a, buf_b, aw_a, aw_b, ij_a, ij_b, id_a, id_b, iw_a, iw_b,
               den_t, mv, sem_sm, sem_rows, sem_acc, acc_sh):
    cid = lax.axis_index("c")
    sid = lax.axis_index("s")
    base0 = sid * EPT2

    slot_a = (buf_a, aw_a, ij_a, id_a, iw_a)
    slot_b = (buf_b, aw_b, ij_b, id_b, iw_b)

    pltpu.sync_copy(m_arr.at[pl.ds(0, LANES)], mv)
    zv = jnp.zeros((LANES,), jnp.float32)

    def zrow(r, carry):
        for c in range(DH // LANES):
            buf_a[r, pl.ds(c * LANES, LANES)] = zv
            buf_b[r, pl.ds(c * LANES, LANES)] = zv
        return carry

    lax.fori_loop(0, CH2, zrow, 0)

    def zden(g, carry):
        den_t[pl.ds(g * LANES, LANES)] = zv
        return carry

    lax.fori_loop(0, N // LANES, zden, 0)
    rows0 = sid * ROWA
    pltpu.sync_copy(buf_a, acc_sh.at[pl.ds(rows0, CH2)])
    pltpu.sync_copy(buf_a.at[pl.ds(0, ROWA - CH2)],
                    acc_sh.at[pl.ds(rows0 + CH2, ROWA - CH2)])

    @pl.when(sid == NS - 1)
    def _():
        pltpu.sync_copy(buf_a.at[pl.ds(0, TAIL)], acc_sh.at[pl.ds(NS * ROWA, TAIL)])

    plsc.subcore_barrier()

    def issue_smalls(slot, c):
        _, aw_, ij_, id_, iw_ = slot
        base = base0 + c * CH2
        rw0 = sid * (EPT2 // SUB) + c * NSUB
        pltpu.async_copy(src.at[pl.ds(base, CH2)], ij_, sem_sm)
        pltpu.async_copy(alpha.at[pl.ds(base, CH2)], aw_, sem_sm)
        pltpu.async_copy(dst.at[pl.ds(base, CH2)], id_, sem_sm)
        pltpu.async_copy(dst2d.at[pl.ds(rw0, NSUB)], iw_, sem_sm)

    def wait_smalls(slot):
        _, aw_, ij_, id_, iw_ = slot
        pltpu.make_async_copy(src.at[pl.ds(base0, CH2)], ij_, sem_sm).wait()
        pltpu.make_async_copy(alpha.at[pl.ds(base0, CH2)], aw_, sem_sm).wait()
        pltpu.make_async_copy(dst.at[pl.ds(base0, CH2)], id_, sem_sm).wait()
        pltpu.make_async_copy(dst2d.at[pl.ds(0, NSUB)], iw_, sem_sm).wait()

    def issue_rows(slot):
        buf_, _, ij_, _, _ = slot

        @pl.when(cid == 0)
        def _():
            pltpu.async_copy(nf_lo.at[ij_], buf_, sem_rows)

        @pl.when(cid == 1)
        def _():
            pltpu.async_copy(nf_hi.at[ij_], buf_, sem_rows)

    def wait_rows(slot):
        buf_, _, ij_, _, _ = slot
        pltpu.make_async_copy(nf_lo.at[ij_], buf_, sem_rows).wait()

    def compute(slot):
        buf_, aw_, _, id_, _ = slot

        def grp(g, carry2):
            av = aw_[pl.ds(g * LANES, LANES)]
            ex = jnp.exp(av - mv[...])
            di = id_[pl.ds(g * LANES, LANES)]
            plsc.addupdate_scatter(den_t, [di], ex)
            for l in range(LANES):
                b = jnp.broadcast_to(ex[l], (LANES,))
                r = g * LANES + l
                for c2_ in range(DH // LANES):
                    sl = pl.ds(c2_ * LANES, LANES)
                    buf_[r, sl] = buf_[r, sl] * b
            return carry2

        lax.fori_loop(0, CH2 // LANES, grp, 0)

    def issue_scatter(slot):
        buf_, _, _, _, iw_ = slot
        for t in range(NSUB):
            pltpu.async_copy(buf_.at[pl.ds(t * SUB, SUB)],
                             acc_sh.at[iw_.at[t]], sem_acc, add=True)

    def wait_scatter(slot):
        buf_, _, _, _, iw_ = slot
        for t in range(NSUB):
            pltpu.make_async_copy(buf_.at[pl.ds(t * SUB, SUB)],
                                  acc_sh.at[iw_.at[t]], sem_acc).wait()

    issue_smalls(slot_a, 0)
    wait_smalls(slot_a)
    issue_rows(slot_a)

    def body(it, carry):
        a = 2 * it
        wait_rows(slot_a)

        @pl.when(it > 0)
        def _():
            wait_scatter(slot_b)

        issue_smalls(slot_b, a + 1)
        compute(slot_a)
        wait_smalls(slot_b)
        issue_rows(slot_b)
        issue_scatter(slot_a)
        wait_rows(slot_b)
        wait_scatter(slot_a)

        @pl.when(it < NIT - 1)
        def _():
            issue_smalls(slot_a, a + 2)

        compute(slot_b)

        @pl.when(it < NIT - 1)
        def _():
            wait_smalls(slot_a)
            issue_rows(slot_a)

        issue_scatter(slot_b)
        return carry

    lax.fori_loop(0, NIT, body, 0)
    wait_scatter(slot_b)
    plsc.subcore_barrier()

    pltpu.sync_copy(acc_sh.at[pl.ds(rows0, ROWA)],
                    acc_out.at[cid, pl.ds(rows0, ROWA)])

    @pl.when(cid == 0)
    def _():
        pltpu.sync_copy(den_t, den_out.at[sid])

    @pl.when(sid == NS - 1)
    def _():
        pltpu.sync_copy(acc_sh.at[pl.ds(NS * ROWA, TAIL)],
                        acc_out.at[cid, pl.ds(NS * ROWA, TAIL)])


BN = 2000


def _combine_body(nf, acc, den, out):
    a = jnp.concatenate([acc[0], acc[1]], axis=-1)
    d = jnp.sum(den[...], axis=1)[:, None]
    d = jnp.where(d > 0.0, d, 1.0)
    out[...] = nf[...] + a / d


def _run_combine(node_feature, acc, den):
    return pl.pallas_call(
        _combine_body,
        grid=(N // BN,),
        in_specs=[
            pl.BlockSpec((BN, D), lambda i: (i, 0)),
            pl.BlockSpec((NC, BN, DH), lambda i: (0, i, 0)),
            pl.BlockSpec((BN, NS), lambda i: (i, 0)),
        ],
        out_specs=pl.BlockSpec((BN, D), lambda i: (i, 0)),
        out_shape=jax.ShapeDtypeStruct((N, D), jnp.float32),
    )(node_feature, acc, den)


def kernel(node_feature, geo_encoding, edge_index, is_source,
           W0, b0, W1, b1, W2, b2, W3, b3, att):
    del is_source
    bf = jnp.bfloat16
    w0t = W0.T.astype(bf)
    w0a = w0t[:D]
    w0b = w0t[D:2 * D]
    w0c = w0t[2 * D:]
    ones = jnp.ones((D, D), bf)
    src_idx = edge_index[0]
    dst_idx = edge_index[1]
    dst2d = dst_idx.reshape(E // SUB, SUB)
    geot = geo_encoding.T
    nf_lo = node_feature[:, :DH]
    nf_hi = node_feature[:, DH:]
    fi_g, fj_g = _gather_k(node_feature, dst_idx, src_idx)
    alpha3, m2d = _run_mlp(
        fi_g, fj_g, geot, w0a, w0b, w0c, b0.reshape(1, D),
        W1.T.astype(bf), b1.reshape(1, D), W2.T.astype(bf), b2.reshape(1, D),
        W3.T.astype(bf), b3.reshape(1, D), att, ones,
    )
    alpha = alpha3.reshape(E)
    m_arr = m2d.reshape(D)
    acc, den = _scatter_k(nf_lo, nf_hi, alpha, m_arr, dst2d, src_idx, dst_idx)
    return _run_combine(node_feature, acc, den.T)


# speedup vs baseline: 6.0880x; 1.0182x over previous
import functools

import jax
import jax.numpy as jnp
from jax import lax
from jax.experimental import pallas as pl
from jax.experimental.pallas import tpu as pltpu
from jax.experimental.pallas import tpu_sc as plsc

N = 10000
E = 320000
D = 128
G = 13
DH = D // 2

NC = 2
NS = 16
LANES = 16
NW = NC * NS
EPT = E // NW
EPT2 = E // NS
CH = 200
CH2 = 400
SUB = 100
NSUB = CH2 // SUB
NIT = EPT2 // CH2 // 2
ROWA = 624
TAIL = N - NS * ROWA

_MESH = plsc.VectorSubcoreMesh(
    core_axis_name="c", subcore_axis_name="s", num_cores=NC, num_subcores=NS
)
_SC_LINEAR = pltpu.CompilerParams(use_tc_tiling_on_sc=False,
                                  needs_layout_passes=False)


@functools.partial(
    pl.kernel,
    out_type=(
        jax.ShapeDtypeStruct((E, D), jnp.float32),
        jax.ShapeDtypeStruct((E, D), jnp.float32),
    ),
    mesh=_MESH,
    scratch_types=[
        pltpu.VMEM((CH,), jnp.int32),
        pltpu.VMEM((CH,), jnp.int32),
        pltpu.VMEM((CH,), jnp.int32),
        pltpu.VMEM((CH,), jnp.int32),
        pltpu.VMEM((CH, D), jnp.float32),
        pltpu.VMEM((CH, D), jnp.float32),
        pltpu.VMEM((CH, D), jnp.float32),
        pltpu.VMEM((CH, D), jnp.float32),
        pltpu.SemaphoreType.DMA,
        pltpu.SemaphoreType.DMA,
        pltpu.SemaphoreType.DMA,
    ],
)
def _gather_k(nf, dst, src, fi_out, fj_out, ii_a, ij_a, ii_b, ij_b,
              bi_a, bj_a, bi_b, bj_b, sem_ix, sem_g, sem_wr):
    wid = lax.axis_index("s") * NC + lax.axis_index("c")
    base0 = wid * EPT
    slot_a = (ii_a, ij_a, bi_a, bj_a)
    slot_b = (ii_b, ij_b, bi_b, bj_b)
    nchunk = EPT // CH

    def issue_idx(slot, c):
        ii, ij, _, _ = slot
        base = base0 + c * CH
        pltpu.async_copy(dst.at[pl.ds(base, CH)], ii, sem_ix)
        pltpu.async_copy(src.at[pl.ds(base, CH)], ij, sem_ix)

    def wait_idx(slot):
        ii, ij, _, _ = slot
        pltpu.make_async_copy(dst.at[pl.ds(base0, CH)], ii, sem_ix).wait()
        pltpu.make_async_copy(src.at[pl.ds(base0, CH)], ij, sem_ix).wait()

    def issue_rows(slot):
        ii, ij, bi, bj = slot
        pltpu.async_copy(nf.at[ii], bi, sem_g)
        pltpu.async_copy(nf.at[ij], bj, sem_g)

    def wait_rows(slot):
        ii, ij, bi, bj = slot
        pltpu.make_async_copy(nf.at[ii], bi, sem_g).wait()
        pltpu.make_async_copy(nf.at[ij], bj, sem_g).wait()

    def issue_wr(slot, c):
        _, _, bi, bj = slot
        base = base0 + c * CH
        pltpu.async_copy(bi, fi_out.at[pl.ds(base, CH)], sem_wr)
        pltpu.async_copy(bj, fj_out.at[pl.ds(base, CH)], sem_wr)

    def wait_wr(slot):
        _, _, bi, bj = slot
        pltpu.make_async_copy(bi, fi_out.at[pl.ds(base0, CH)], sem_wr).wait()
        pltpu.make_async_copy(bj, fj_out.at[pl.ds(base0, CH)], sem_wr).wait()

    issue_idx(slot_a, 0)
    wait_idx(slot_a)
    issue_rows(slot_a)

    def body(it, carry):
        a = 2 * it
        issue_idx(slot_b, a + 1)
        wait_rows(slot_a)
        wait_idx(slot_b)

        @pl.when(it > 0)
        def _():
            wait_wr(slot_b)

        issue_rows(slot_b)
        issue_wr(slot_a, a)

        @pl.when(it < nchunk // 2 - 1)
        def _():
            issue_idx(slot_a, a + 2)

        wait_rows(slot_b)
        wait_wr(slot_a)

        @pl.when(it < nchunk // 2 - 1)
        def _():
            wait_idx(slot_a)
            issue_rows(slot_a)

        issue_wr(slot_b, a + 1)
        return carry

    lax.fori_loop(0, nchunk // 2, body, 0)
    wait_wr(slot_b)


BLK = 2560
NBLK = E // BLK


def _mlp_body(fi, fj, geot, w0a, w0b, w0c, b0, w1, b1, w2, b2, w3, b3,
              att, ones, alpha_ref, m_ref, mscr):
    step = pl.program_id(0)

    @pl.when(step == 0)
    def _():
        mscr[0] = -jnp.inf

    bf = jnp.bfloat16
    f32 = jnp.float32

    def mm(a, w):
        return jnp.dot(a.astype(bf), w[...], preferred_element_type=f32)

    hg = lax.dot_general(geot[...].astype(bf), w0c[...],
                         (((0,), (0,)), ((), ())),
                         preferred_element_type=f32)
    h = mm(fi[...], w0a) + mm(fj[...], w0b) + hg + b0[...]
    h = jnp.maximum(h, 0.0)
    h = jnp.maximum(mm(h, w1) + b1[...], 0.0)
    h = jnp.maximum(mm(h, w2) + b2[...], 0.0)
    h = jnp.maximum(mm(h, w3) + b3[...], 0.0)
    y = h * att[...]
    y = jnp.where(y >= 0.0, y, 0.01 * y)
    zt = lax.dot_general(ones[...], y.astype(bf), (((0,), (1,)), ((), ())),
                         preferred_element_type=f32)
    alpha_ref[0, 0, :] = zt[0]
    mscr[0] = jnp.maximum(mscr[0], jnp.max(zt[0]))

    @pl.when(step == NBLK - 1)
    def _():
        m_ref[...] = jnp.full((1, D), mscr[0], jnp.float32)


def _run_mlp(fi_g, fj_g, geot, w0a, w0b, w0c, b0, w1, b1, w2, b2, w3, b3,
             att, ones):
    full = lambda shape: pl.BlockSpec(shape, lambda i: (0,) * len(shape))
    alpha3, m2d = pl.pallas_call(
        _mlp_body,
        grid=(NBLK,),
        in_specs=[
            pl.BlockSpec((BLK, D), lambda i: (i, 0)),
            pl.BlockSpec((BLK, D), lambda i: (i, 0)),
            pl.BlockSpec((G, BLK), lambda i: (0, i)),
            full((D, D)), full((D, D)), full((G, D)), full((1, D)),
            full((D, D)), full((1, D)),
            full((D, D)), full((1, D)),
            full((D, D)), full((1, D)),
            full((1, D)), full((D, D)),
        ],
        out_specs=[
            pl.BlockSpec((1, 1, BLK), lambda i: (i, 0, 0)),
            pl.BlockSpec((1, D), lambda i: (0, 0)),
        ],
        out_shape=[
            jax.ShapeDtypeStruct((NBLK, 1, BLK), jnp.float32),
            jax.ShapeDtypeStruct((1, D), jnp.float32),
        ],
        scratch_shapes=[pltpu.SMEM((1,), jnp.float32)],
    )(fi_g, fj_g, geot, w0a, w0b, w0c, b0, w1, b1, w2, b2, w3, b3, att, ones)
    return alpha3, m2d


@functools.partial(
    pl.kernel,
    out_type=(
        jax.ShapeDtypeStruct((NC, N, DH), jnp.float32),
        jax.ShapeDtypeStruct((NS, N), jnp.float32),
    ),
    mesh=_MESH,
    scratch_types=[
        pltpu.VMEM((CH2, DH), jnp.float32),
        pltpu.VMEM((CH2, DH), jnp.float32),
        pltpu.VMEM((CH2,), jnp.float32),
        pltpu.VMEM((CH2,), jnp.float32),
        pltpu.VMEM((CH2,), jnp.int32),
        pltpu.VMEM((CH2,), jnp.int32),
        pltpu.VMEM((CH2,), jnp.int32),
        pltpu.VMEM((CH2,), jnp.int32),
        pltpu.VMEM((NSUB, SUB), jnp.int32),
        pltpu.VMEM((NSUB, SUB), jnp.int32),
        pltpu.VMEM((N,), jnp.float32),
        pltpu.VMEM((LANES,), jnp.float32),
        pltpu.SemaphoreType.DMA,
        pltpu.SemaphoreType.DMA,
        pltpu.SemaphoreType.DMA,
        pltpu.VMEM_SHARED((N, DH), jnp.float32),
    ],
    compiler_params=_SC_LINEAR,
)
def _scatter_k(nf_lo, nf_hi, alpha, m_arr, dst2d, src, dst, acc_out, den_out,
               buf_a, buf_b, aw_a, aw_b, ij_a, ij_b, id_a, id_b, iw_a, iw_b,
               den_t, mv, sem_sm, sem_rows, sem_acc, acc_sh):
    cid = lax.axis_index("c")
    sid = lax.axis_index("s")
    base0 = sid * EPT2

    slot_a = (buf_a, aw_a, ij_a, id_a, iw_a)
    slot_b = (buf_b, aw_b, ij_b, id_b, iw_b)

    pltpu.sync_copy(m_arr.at[pl.ds(0, LANES)], mv)
    zv = jnp.zeros((LANES,), jnp.float32)

    def zrow(r, carry):
        for c in range(DH // LANES):
            buf_a[r, pl.ds(c * LANES, LANES)] = zv
            buf_b[r, pl.ds(c * LANES, LANES)] = zv
        return carry

    lax.fori_loop(0, CH2, zrow, 0)

    def zden(g, carry):
        den_t[pl.ds(g * LANES, LANES)] = zv
        return carry

    lax.fori_loop(0, N // LANES, zden, 0)
    rows0 = sid * ROWA
    pltpu.sync_copy(buf_a, acc_sh.at[pl.ds(rows0, CH2)])
    pltpu.sync_copy(buf_a.at[pl.ds(0, ROWA - CH2)],
                    acc_sh.at[pl.ds(rows0 + CH2, ROWA - CH2)])

    @pl.when(sid == NS - 1)
    def _():
        pltpu.sync_copy(buf_a.at[pl.ds(0, TAIL)], acc_sh.at[pl.ds(NS * ROWA, TAIL)])

    plsc.subcore_barrier()

    def issue_smalls(slot, c):
        _, aw_, ij_, id_, iw_ = slot
        base = base0 + c * CH2
        rw0 = sid * (EPT2 // SUB) + c * NSUB
        pltpu.async_copy(src.at[pl.ds(base, CH2)], ij_, sem_sm)
        pltpu.async_copy(alpha.at[pl.ds(base, CH2)], aw_, sem_sm)
        pltpu.async_copy(dst.at[pl.ds(base, CH2)], id_, sem_sm)
        pltpu.async_copy(dst2d.at[pl.ds(rw0, NSUB)], iw_, sem_sm)

    def wait_smalls(slot):
        _, aw_, ij_, id_, iw_ = slot
        pltpu.make_async_copy(src.at[pl.ds(base0, CH2)], ij_, sem_sm).wait()
        pltpu.make_async_copy(alpha.at[pl.ds(base0, CH2)], aw_, sem_sm).wait()
        pltpu.make_async_copy(dst.at[pl.ds(base0, CH2)], id_, sem_sm).wait()
        pltpu.make_async_copy(dst2d.at[pl.ds(0, NSUB)], iw_, sem_sm).wait()

    def issue_rows(slot):
        buf_, _, ij_, _, _ = slot

        @pl.when(cid == 0)
        def _():
            pltpu.async_copy(nf_lo.at[ij_], buf_, sem_rows)

        @pl.when(cid == 1)
        def _():
            pltpu.async_copy(nf_hi.at[ij_], buf_, sem_rows)

    def wait_rows(slot):
        buf_, _, ij_, _, _ = slot
        pltpu.make_async_copy(nf_lo.at[ij_], buf_, sem_rows).wait()

    def compute(slot):
        buf_, aw_, _, id_, _ = slot

        def grp(g, carry2):
            av = aw_[pl.ds(g * LANES, LANES)]
            ex = jnp.exp(av - mv[...])
            di = id_[pl.ds(g * LANES, LANES)]
            plsc.addupdate_scatter(den_t, [di], ex)
            for l in range(LANES):
                b = jnp.broadcast_to(ex[l], (LANES,))
                r = g * LANES + l
                for c2_ in range(DH // LANES):
                    sl = pl.ds(c2_ * LANES, LANES)
                    buf_[r, sl] = buf_[r, sl] * b
            return carry2

        lax.fori_loop(0, CH2 // LANES, grp, 0)

    def issue_scatter(slot):
        buf_, _, _, _, iw_ = slot
        for t in range(NSUB):
            pltpu.async_copy(buf_.at[pl.ds(t * SUB, SUB)],
                             acc_sh.at[iw_.at[t]], sem_acc, add=True)

    def wait_scatter(slot):
        buf_, _, _, _, iw_ = slot
        for t in range(NSUB):
            pltpu.make_async_copy(buf_.at[pl.ds(t * SUB, SUB)],
                                  acc_sh.at[iw_.at[t]], sem_acc).wait()

    issue_smalls(slot_a, 0)
    wait_smalls(slot_a)
    issue_rows(slot_a)

    def body(it, carry):
        a = 2 * it
        wait_rows(slot_a)

        @pl.when(it > 0)
        def _():
            wait_scatter(slot_b)

        issue_smalls(slot_b, a + 1)
        compute(slot_a)
        wait_smalls(slot_b)
        issue_rows(slot_b)
        issue_scatter(slot_a)
        wait_rows(slot_b)
        wait_scatter(slot_a)

        @pl.when(it < NIT - 1)
        def _():
            issue_smalls(slot_a, a + 2)

        compute(slot_b)

        @pl.when(it < NIT - 1)
        def _():
            wait_smalls(slot_a)
            issue_rows(slot_a)

        issue_scatter(slot_b)
        return carry

    lax.fori_loop(0, NIT, body, 0)
    wait_scatter(slot_b)
    plsc.subcore_barrier()

    pltpu.sync_copy(acc_sh.at[pl.ds(rows0, ROWA)],
                    acc_out.at[cid, pl.ds(rows0, ROWA)])

    @pl.when(cid == 0)
    def _():
        pltpu.sync_copy(den_t, den_out.at[sid])

    @pl.when(sid == NS - 1)
    def _():
        pltpu.sync_copy(acc_sh.at[pl.ds(NS * ROWA, TAIL)],
                        acc_out.at[cid, pl.ds(NS * ROWA, TAIL)])


BN = 2000


def _combine_body(nf, acc, den, out):
    a = jnp.concatenate([acc[0], acc[1]], axis=-1)
    d = jnp.sum(den[...], axis=1)[:, None]
    d = jnp.where(d > 0.0, d, 1.0)
    out[...] = nf[...] + a / d


def _run_combine(node_feature, acc, den):
    return pl.pallas_call(
        _combine_body,
        grid=(N // BN,),
        in_specs=[
            pl.BlockSpec((BN, D), lambda i: (i, 0)),
            pl.BlockSpec((NC, BN, DH), lambda i: (0, i, 0)),
            pl.BlockSpec((BN, NS), lambda i: (i, 0)),
        ],
        out_specs=pl.BlockSpec((BN, D), lambda i: (i, 0)),
        out_shape=jax.ShapeDtypeStruct((N, D), jnp.float32),
    )(node_feature, acc, den)


def kernel(node_feature, geo_encoding, edge_index, is_source,
           W0, b0, W1, b1, W2, b2, W3, b3, att):
    del is_source
    bf = jnp.bfloat16
    w0t = W0.T.astype(bf)
    w0a = w0t[:D]
    w0b = w0t[D:2 * D]
    w0c = w0t[2 * D:]
    ones = jnp.ones((D, D), bf)
    src_idx = edge_index[0]
    dst_idx = edge_index[1]
    dst2d = dst_idx.reshape(E // SUB, SUB)
    geot = geo_encoding.T
    nf_lo = node_feature[:, :DH]
    nf_hi = node_feature[:, DH:]
    fi_g, fj_g = _gather_k(node_feature, dst_idx, src_idx)
    alpha3, m2d = _run_mlp(
        fi_g, fj_g, geot, w0a, w0b, w0c, b0.reshape(1, D),
        W1.T.astype(bf), b1.reshape(1, D), W2.T.astype(bf), b2.reshape(1, D),
        W3.T.astype(bf), b3.reshape(1, D), att, ones,
    )
    alpha = alpha3.reshape(E)
    m_arr = m2d.reshape(D)
    acc, den = _scatter_k(nf_lo, nf_hi, alpha, m_arr, dst2d, src_idx, dst_idx)
    return _run_combine(node_feature, acc, den.T)
